# Initial kernel scaffold; baseline (speedup 1.0000x reference)
import functools

import jax
import jax.numpy as jnp
from jax import lax
from jax.experimental import pallas as pl
from jax.experimental.pallas import tpu as pltpu
from jax.experimental.pallas import tpu_sc as plsc

N = 10000
D = 128
E = 320000

NC = 2
NS = 16
NW = NC * NS

CK = 128
EPAD = 327680
EW = EPAD // NW
CHUNKS = EW // CK
NPAD = 10016
ZROWS = NPAD // NS
DROWS = N // NS
DEGW = 16

MBLK = 1000
GRID = N // MBLK

_mesh = plsc.VectorSubcoreMesh(core_axis_name="c", subcore_axis_name="s")


@functools.partial(
    pl.kernel,
    out_type=jax.ShapeDtypeStruct((NC, NPAD, DEGW), jnp.float32),
    mesh=_mesh,
    scratch_types=[
        pltpu.VMEM_SHARED((NPAD, DEGW), jnp.float32),
        pltpu.VMEM((CK,), jnp.int32),
        pltpu.VMEM((CK, DEGW), jnp.float32),
    ],
)
def _deg_kernel(dst_hbm, zeros_hbm, ones_hbm, out_hbm, deg_sh, dst_v, ones_v):
    cid = lax.axis_index("c")
    sid = lax.axis_index("s")
    wid = cid * NS + sid
    pltpu.sync_copy(zeros_hbm, deg_sh.at[pl.ds(sid * ZROWS, ZROWS)])
    pltpu.sync_copy(ones_hbm, ones_v)
    plsc.subcore_barrier()
    base = wid * EW

    def chunk(i, carry):
        off = base + i * CK
        pltpu.sync_copy(dst_hbm.at[pl.ds(off, CK)], dst_v)
        pltpu.sync_copy(ones_v, deg_sh.at[dst_v], add=True)
        return carry

    lax.fori_loop(0, CHUNKS, chunk, 0)
    plsc.subcore_barrier()
    pltpu.sync_copy(deg_sh.at[pl.ds(sid * ZROWS, ZROWS)],
                    out_hbm.at[cid, pl.ds(sid * ZROWS, ZROWS)])


@functools.partial(
    pl.kernel,
    out_type=jax.ShapeDtypeStruct((NC, N, D), jnp.float32),
    mesh=_mesh,
    scratch_types=[
        pltpu.VMEM_SHARED((NPAD, D), jnp.float32),
        pltpu.VMEM((CK,), jnp.int32),
        pltpu.VMEM((CK,), jnp.int32),
        pltpu.VMEM((CK, D), jnp.float32),
        pltpu.SemaphoreType.DMA,
    ],
)
def _agg_kernel(y_hbm, src_hbm, dst_hbm, zeros_hbm, out_hbm,
                agg_sh, src_v, dst_v, rows_v, sem):
    cid = lax.axis_index("c")
    sid = lax.axis_index("s")
    wid = cid * NS + sid
    pltpu.sync_copy(zeros_hbm, agg_sh.at[pl.ds(sid * ZROWS, ZROWS)])
    plsc.subcore_barrier()
    base = wid * EW

    def chunk(i, carry):
        off = base + i * CK
        pltpu.sync_copy(src_hbm.at[pl.ds(off, CK)], src_v)
        pltpu.sync_copy(dst_hbm.at[pl.ds(off, CK)], dst_v)
        pltpu.async_copy(y_hbm.at[src_v], rows_v, sem).wait()
        pltpu.sync_copy(rows_v, agg_sh.at[dst_v], add=True)
        return carry

    lax.fori_loop(0, CHUNKS, chunk, 0)
    plsc.subcore_barrier()
    pltpu.sync_copy(agg_sh.at[pl.ds(sid * DROWS, DROWS)],
                    out_hbm.at[cid, pl.ds(sid * DROWS, DROWS)])


def _pre_body(x_ref, w_ref, deg_ref, dinv_ref, y_ref):
    deg = deg_ref[0, :, :1] + deg_ref[1, :, :1] + 1.0
    dinv = lax.rsqrt(deg)
    dinv_ref[...] = dinv
    y_ref[...] = dinv * jnp.dot(x_ref[...], w_ref[...],
                                preferred_element_type=jnp.float32)


def _pre_call(x, w0, deg):
    return pl.pallas_call(
        _pre_body,
        grid=(GRID,),
        in_specs=[
            pl.BlockSpec((MBLK, D), lambda i: (i, 0)),
            pl.BlockSpec((D, D), lambda i: (0, 0)),
            pl.BlockSpec((NC, MBLK, DEGW), lambda i: (0, i, 0)),
        ],
        out_specs=[
            pl.BlockSpec((MBLK, 1), lambda i: (i, 0)),
            pl.BlockSpec((MBLK, D), lambda i: (i, 0)),
        ],
        out_shape=[
            jax.ShapeDtypeStruct((N, 1), jnp.float32),
            jax.ShapeDtypeStruct((N, D), jnp.float32),
        ],
    )(x, w0, deg)


def _mid_body(agg_ref, y_ref, dinv_ref, w_ref, b_ref, y1_ref):
    dinv = dinv_ref[...]
    h = dinv * (agg_ref[0] + agg_ref[1] + y_ref[...]) + b_ref[...]
    y1_ref[...] = dinv * jnp.dot(h, w_ref[...],
                                 preferred_element_type=jnp.float32)


def _mid_call(agg, y0, dinv, w1, b0):
    return pl.pallas_call(
        _mid_body,
        grid=(GRID,),
        in_specs=[
            pl.BlockSpec((NC, MBLK, D), lambda i: (0, i, 0)),
            pl.BlockSpec((MBLK, D), lambda i: (i, 0)),
            pl.BlockSpec((MBLK, 1), lambda i: (i, 0)),
            pl.BlockSpec((D, D), lambda i: (0, 0)),
            pl.BlockSpec((1, D), lambda i: (0, 0)),
        ],
        out_specs=pl.BlockSpec((MBLK, D), lambda i: (i, 0)),
        out_shape=jax.ShapeDtypeStruct((N, D), jnp.float32),
    )(agg, y0, dinv, w1, b0)


def _post_body(agg_ref, y_ref, dinv_ref, b_ref, out_ref):
    out_ref[...] = (dinv_ref[...] * (agg_ref[0] + agg_ref[1] + y_ref[...])
                    + b_ref[...])


def _post_call(agg, y1, dinv, b1):
    return pl.pallas_call(
        _post_body,
        grid=(GRID,),
        in_specs=[
            pl.BlockSpec((NC, MBLK, D), lambda i: (0, i, 0)),
            pl.BlockSpec((MBLK, D), lambda i: (i, 0)),
            pl.BlockSpec((MBLK, 1), lambda i: (i, 0)),
            pl.BlockSpec((1, D), lambda i: (0, 0)),
        ],
        out_specs=pl.BlockSpec((MBLK, D), lambda i: (i, 0)),
        out_shape=jax.ShapeDtypeStruct((N, D), jnp.float32),
    )(agg, y1, dinv, b1)


def kernel(nodes_embeddings, edges, W0, b0, W1, b1):
    src = jnp.concatenate(
        [edges[0].astype(jnp.int32), jnp.zeros((EPAD - E,), jnp.int32)])
    dst = jnp.concatenate(
        [edges[1].astype(jnp.int32), jnp.full((EPAD - E,), N, jnp.int32)])

    zeros_deg = jnp.zeros((ZROWS, DEGW), jnp.float32)
    ones_deg = jnp.ones((CK, DEGW), jnp.float32)
    zeros_agg = jnp.zeros((ZROWS, D), jnp.float32)
    b0r = b0.reshape(1, D)
    b1r = b1.reshape(1, D)

    deg = _deg_kernel(dst, zeros_deg, ones_deg)

    dinv, y0 = _pre_call(nodes_embeddings, W0, deg)
    agg0 = _agg_kernel(y0, src, dst, zeros_agg)
    y1 = _mid_call(agg0, y0, dinv, W1, b0r)
    agg1 = _agg_kernel(y1, src, dst, zeros_agg)
    return _post_call(agg1, y1, dinv, b1r)


# speedup vs baseline: 7.5225x; 7.5225x over previous
import functools

import jax
import jax.numpy as jnp
from jax import lax
from jax.experimental import pallas as pl
from jax.experimental.pallas import tpu as pltpu
from jax.experimental.pallas import tpu_sc as plsc

N = 10000
D = 128
E = 320000

NC = 2
NS = 16
NW = NC * NS

CK = 128
EPAD = 327680
EW = EPAD // NW
CHUNKS = EW // CK
NPAD = 10112
ZROWS = NPAD // NS
DEGW = 128

MBLK = 1000
GRID = N // MBLK


@functools.lru_cache(maxsize=None)
def _sc_kernels():
    mesh = plsc.VectorSubcoreMesh(core_axis_name="c", subcore_axis_name="s",
                                  num_cores=NC, num_subcores=NS)

    @functools.partial(
        pl.kernel,
        out_type=jax.ShapeDtypeStruct((NC, NPAD, DEGW), jnp.float32),
        mesh=mesh,
        scratch_types=[
            pltpu.VMEM_SHARED((NPAD, DEGW), jnp.float32),
            pltpu.VMEM((CK,), jnp.int32),
            pltpu.VMEM((CK, DEGW), jnp.float32),
        ],
    )
    def deg_kernel(dst_hbm, zeros_hbm, ones_hbm, out_hbm, deg_sh, dst_v, ones_v):
        cid = lax.axis_index("c")
        sid = lax.axis_index("s")
        wid = cid * NS + sid
        pltpu.sync_copy(zeros_hbm, deg_sh.at[pl.ds(sid * ZROWS, ZROWS)])
        pltpu.sync_copy(ones_hbm, ones_v)
        plsc.subcore_barrier()
        base = wid * EW

        def chunk(i, carry):
            off = base + i * CK
            pltpu.sync_copy(dst_hbm.at[pl.ds(off, CK)], dst_v)
            pltpu.sync_copy(ones_v, deg_sh.at[dst_v], add=True)
            return carry

        lax.fori_loop(0, CHUNKS, chunk, 0)
        plsc.subcore_barrier()
        pltpu.sync_copy(deg_sh.at[pl.ds(sid * ZROWS, ZROWS)],
                        out_hbm.at[cid, pl.ds(sid * ZROWS, ZROWS)])

    @functools.partial(
        pl.kernel,
        out_type=jax.ShapeDtypeStruct((NC, NPAD, D), jnp.float32),
        mesh=mesh,
        scratch_types=[
            pltpu.VMEM_SHARED((NPAD, D), jnp.float32),
            pltpu.VMEM((CK,), jnp.int32),
            pltpu.VMEM((CK,), jnp.int32),
            pltpu.VMEM((CK, D), jnp.float32),
            pltpu.SemaphoreType.DMA,
        ],
    )
    def agg_kernel(y_hbm, src_hbm, dst_hbm, zeros_hbm, out_hbm,
                   agg_sh, src_v, dst_v, rows_v, sem):
        cid = lax.axis_index("c")
        sid = lax.axis_index("s")
        wid = cid * NS + sid
        pltpu.sync_copy(zeros_hbm, agg_sh.at[pl.ds(sid * ZROWS, ZROWS)])
        plsc.subcore_barrier()
        base = wid * EW

        def chunk(i, carry):
            off = base + i * CK
            pltpu.sync_copy(src_hbm.at[pl.ds(off, CK)], src_v)
            pltpu.sync_copy(dst_hbm.at[pl.ds(off, CK)], dst_v)
            pltpu.async_copy(y_hbm.at[src_v], rows_v, sem).wait()
            pltpu.sync_copy(rows_v, agg_sh.at[dst_v], add=True)
            return carry

        lax.fori_loop(0, CHUNKS, chunk, 0)
        plsc.subcore_barrier()
        pltpu.sync_copy(agg_sh.at[pl.ds(sid * ZROWS, ZROWS)],
                        out_hbm.at[cid, pl.ds(sid * ZROWS, ZROWS)])

    return deg_kernel, agg_kernel


def _pre_body(x_ref, w_ref, deg_ref, dinv_ref, y_ref):
    deg = deg_ref[0, :, :1] + deg_ref[1, :, :1] + 1.0
    dinv = lax.rsqrt(deg)
    dinv_ref[...] = dinv
    y_ref[...] = dinv * jnp.dot(x_ref[...], w_ref[...],
                                preferred_element_type=jnp.float32)


def _pre_call(x, w0, deg):
    return pl.pallas_call(
        _pre_body,
        grid=(GRID,),
        in_specs=[
            pl.BlockSpec((MBLK, D), lambda i: (i, 0)),
            pl.BlockSpec((D, D), lambda i: (0, 0)),
            pl.BlockSpec((NC, MBLK, DEGW), lambda i: (0, i, 0)),
        ],
        out_specs=[
            pl.BlockSpec((MBLK, 1), lambda i: (i, 0)),
            pl.BlockSpec((MBLK, D), lambda i: (i, 0)),
        ],
        out_shape=[
            jax.ShapeDtypeStruct((N, 1), jnp.float32),
            jax.ShapeDtypeStruct((N, D), jnp.float32),
        ],
    )(x, w0, deg)


def _mid_body(agg_ref, y_ref, dinv_ref, w_ref, b_ref, y1_ref):
    dinv = dinv_ref[...]
    h = dinv * (agg_ref[0] + agg_ref[1] + y_ref[...]) + b_ref[...]
    y1_ref[...] = dinv * jnp.dot(h, w_ref[...],
                                 preferred_element_type=jnp.float32)


def _mid_call(agg, y0, dinv, w1, b0):
    return pl.pallas_call(
        _mid_body,
        grid=(GRID,),
        in_specs=[
            pl.BlockSpec((NC, MBLK, D), lambda i: (0, i, 0)),
            pl.BlockSpec((MBLK, D), lambda i: (i, 0)),
            pl.BlockSpec((MBLK, 1), lambda i: (i, 0)),
            pl.BlockSpec((D, D), lambda i: (0, 0)),
            pl.BlockSpec((1, D), lambda i: (0, 0)),
        ],
        out_specs=pl.BlockSpec((MBLK, D), lambda i: (i, 0)),
        out_shape=jax.ShapeDtypeStruct((N, D), jnp.float32),
    )(agg, y0, dinv, w1, b0)


def _post_body(agg_ref, y_ref, dinv_ref, b_ref, out_ref):
    out_ref[...] = (dinv_ref[...] * (agg_ref[0] + agg_ref[1] + y_ref[...])
                    + b_ref[...])


def _post_call(agg, y1, dinv, b1):
    return pl.pallas_call(
        _post_body,
        grid=(GRID,),
        in_specs=[
            pl.BlockSpec((NC, MBLK, D), lambda i: (0, i, 0)),
            pl.BlockSpec((MBLK, D), lambda i: (i, 0)),
            pl.BlockSpec((MBLK, 1), lambda i: (i, 0)),
            pl.BlockSpec((1, D), lambda i: (0, 0)),
        ],
        out_specs=pl.BlockSpec((MBLK, D), lambda i: (i, 0)),
        out_shape=jax.ShapeDtypeStruct((N, D), jnp.float32),
    )(agg, y1, dinv, b1)


def kernel(nodes_embeddings, edges, W0, b0, W1, b1):
    src = jnp.concatenate(
        [edges[0].astype(jnp.int32), jnp.zeros((EPAD - E,), jnp.int32)])
    dst = jnp.concatenate(
        [edges[1].astype(jnp.int32), jnp.full((EPAD - E,), N, jnp.int32)])

    zeros_deg = jnp.zeros((ZROWS, DEGW), jnp.float32)
    ones_deg = jnp.ones((CK, DEGW), jnp.float32)
    zeros_agg = jnp.zeros((ZROWS, D), jnp.float32)
    b0r = b0.reshape(1, D)
    b1r = b1.reshape(1, D)

    deg_kernel, agg_kernel = _sc_kernels()
    deg = deg_kernel(dst, zeros_deg, ones_deg)

    dinv, y0 = _pre_call(nodes_embeddings, W0, deg)
    agg0 = agg_kernel(y0, src, dst, zeros_agg)
    y1 = _mid_call(agg0, y0, dinv, W1, b0r)
    agg1 = agg_kernel(y1, src, dst, zeros_agg)
    return _post_call(agg1, y1, dinv, b1r)


# speedup vs baseline: 9.5043x; 1.2635x over previous
import functools

import jax
import jax.numpy as jnp
from jax import lax
from jax.experimental import pallas as pl
from jax.experimental.pallas import tpu as pltpu
from jax.experimental.pallas import tpu_sc as plsc

N = 10000
D = 128
E = 320000

NC = 2
NS = 16
NW = NC * NS

CK = 128
EPAD = 327680
EW = EPAD // NW
CHUNKS = EW // CK
NPAD = 10112
ZROWS = NPAD // NS
DEGW = 128
NBUF = 4
NIB = 4

MBLK = 1000
GRID = N // MBLK


@functools.lru_cache(maxsize=None)
def _sc_kernels():
    mesh = plsc.VectorSubcoreMesh(core_axis_name="c", subcore_axis_name="s",
                                  num_cores=NC, num_subcores=NS)

    @functools.partial(
        pl.kernel,
        out_type=jax.ShapeDtypeStruct((NC, NPAD, DEGW), jnp.float32),
        mesh=mesh,
        scratch_types=[
            pltpu.VMEM_SHARED((NPAD, DEGW), jnp.float32),
            pltpu.VMEM((CHUNKS, CK), jnp.int32),
            pltpu.VMEM((CK, DEGW), jnp.float32),
            pltpu.SemaphoreType.DMA,
        ],
    )
    def deg_kernel(dst_hbm, zeros_hbm, ones_hbm, out_hbm,
                   deg_sh, dst_buf, ones_v, sem):
        cid = lax.axis_index("c")
        sid = lax.axis_index("s")
        wid = cid * NS + sid
        pltpu.sync_copy(zeros_hbm, deg_sh.at[pl.ds(sid * ZROWS, ZROWS)])
        pltpu.sync_copy(ones_hbm, ones_v)
        pltpu.sync_copy(dst_hbm.at[wid], dst_buf)
        plsc.subcore_barrier()

        def dummy_wait(s):
            pltpu.make_async_copy(ones_hbm, ones_v, s).wait()

        def grp(g, carry):
            for b in range(NBUF):
                c = g * NBUF + b
                pltpu.async_copy(ones_v, deg_sh.at[dst_buf.at[c]], sem,
                                 add=True)

                @pl.when(g > 0)
                def _():
                    dummy_wait(sem)
            return carry

        lax.fori_loop(0, CHUNKS // NBUF, grp, 0)
        for b in range(NBUF):
            dummy_wait(sem)
        plsc.subcore_barrier()
        pltpu.sync_copy(deg_sh.at[pl.ds(sid * ZROWS, ZROWS)],
                        out_hbm.at[cid, pl.ds(sid * ZROWS, ZROWS)])

    @functools.partial(
        pl.kernel,
        out_type=jax.ShapeDtypeStruct((NC, NPAD, D), jnp.float32),
        mesh=mesh,
        scratch_types=[
            pltpu.VMEM_SHARED((NPAD, D), jnp.float32),
            pltpu.VMEM((NIB, CK), jnp.int32),
            pltpu.VMEM((NIB, CK), jnp.int32),
            pltpu.VMEM((2, CK, D), jnp.float32),
            pltpu.SemaphoreType.DMA((NIB,)),
            pltpu.SemaphoreType.DMA((2,)),
            pltpu.SemaphoreType.DMA((2,)),
        ],
    )
    def agg_kernel(y_hbm, src_hbm, dst_hbm, zeros_hbm, out_hbm,
                   agg_sh, src_i, dst_i, rows_v, isem, gsem, ssem):
        cid = lax.axis_index("c")
        sid = lax.axis_index("s")
        wid = cid * NS + sid
        pltpu.sync_copy(zeros_hbm, agg_sh.at[pl.ds(sid * ZROWS, ZROWS)])
        plsc.subcore_barrier()

        def fire_idx(c, islot):
            pltpu.async_copy(src_hbm.at[wid, c], src_i.at[islot],
                             isem.at[islot])
            pltpu.async_copy(dst_hbm.at[wid, c], dst_i.at[islot],
                             isem.at[islot])

        def wait_idx(islot):
            for _ in range(2):
                pltpu.make_async_copy(src_hbm.at[0, 0], src_i.at[islot],
                                      isem.at[islot]).wait()

        def fire_gather(c_islot, rslot):
            pltpu.async_copy(y_hbm.at[src_i.at[c_islot]], rows_v.at[rslot],
                             gsem.at[rslot])

        def wait_rows(rslot, s):
            pltpu.make_async_copy(y_hbm.at[pl.ds(0, CK)], rows_v.at[rslot],
                                  s.at[rslot]).wait()

        for c0 in range(3):
            fire_idx(c0, c0)
        wait_idx(0)
        fire_gather(0, 0)

        def grp(g, carry):
            for b in range(4):
                c = g * 4 + b
                rslot = b % 2
                oslot = 1 - rslot
                wait_rows(rslot, gsem)
                pltpu.async_copy(rows_v.at[rslot],
                                 agg_sh.at[dst_i.at[b]],
                                 ssem.at[rslot], add=True)

                @pl.when(c >= 1)
                def _():
                    wait_rows(oslot, ssem)

                @pl.when(c + 3 < CHUNKS)
                def _():
                    fire_idx(c + 3, (b + 3) % NIB)

                @pl.when(c + 1 < CHUNKS)
                def _():
                    wait_idx((b + 1) % NIB)
                    fire_gather((b + 1) % NIB, oslot)
            return carry

        lax.fori_loop(0, CHUNKS // 4, grp, 0)
        wait_rows((CHUNKS - 1) % 2, ssem)
        plsc.subcore_barrier()
        pltpu.sync_copy(agg_sh.at[pl.ds(sid * ZROWS, ZROWS)],
                        out_hbm.at[cid, pl.ds(sid * ZROWS, ZROWS)])

    return deg_kernel, agg_kernel


def _pre_body(x_ref, w_ref, deg_ref, dinv_ref, y_ref):
    deg = deg_ref[0, :, :1] + deg_ref[1, :, :1] + 1.0
    dinv = lax.rsqrt(deg)
    dinv_ref[...] = dinv
    y_ref[...] = dinv * jnp.dot(x_ref[...], w_ref[...],
                                preferred_element_type=jnp.float32)


def _pre_call(x, w0, deg):
    return pl.pallas_call(
        _pre_body,
        grid=(GRID,),
        in_specs=[
            pl.BlockSpec((MBLK, D), lambda i: (i, 0)),
            pl.BlockSpec((D, D), lambda i: (0, 0)),
            pl.BlockSpec((NC, MBLK, DEGW), lambda i: (0, i, 0)),
        ],
        out_specs=[
            pl.BlockSpec((MBLK, 1), lambda i: (i, 0)),
            pl.BlockSpec((MBLK, D), lambda i: (i, 0)),
        ],
        out_shape=[
            jax.ShapeDtypeStruct((N, 1), jnp.float32),
            jax.ShapeDtypeStruct((N, D), jnp.float32),
        ],
    )(x, w0, deg)


def _mid_body(agg_ref, y_ref, dinv_ref, w_ref, b_ref, y1_ref):
    dinv = dinv_ref[...]
    h = dinv * (agg_ref[0] + agg_ref[1] + y_ref[...]) + b_ref[...]
    y1_ref[...] = dinv * jnp.dot(h, w_ref[...],
                                 preferred_element_type=jnp.float32)


def _mid_call(agg, y0, dinv, w1, b0):
    return pl.pallas_call(
        _mid_body,
        grid=(GRID,),
        in_specs=[
            pl.BlockSpec((NC, MBLK, D), lambda i: (0, i, 0)),
            pl.BlockSpec((MBLK, D), lambda i: (i, 0)),
            pl.BlockSpec((MBLK, 1), lambda i: (i, 0)),
            pl.BlockSpec((D, D), lambda i: (0, 0)),
            pl.BlockSpec((1, D), lambda i: (0, 0)),
        ],
        out_specs=pl.BlockSpec((MBLK, D), lambda i: (i, 0)),
        out_shape=jax.ShapeDtypeStruct((N, D), jnp.float32),
    )(agg, y0, dinv, w1, b0)


def _post_body(agg_ref, y_ref, dinv_ref, b_ref, out_ref):
    out_ref[...] = (dinv_ref[...] * (agg_ref[0] + agg_ref[1] + y_ref[...])
                    + b_ref[...])


def _post_call(agg, y1, dinv, b1):
    return pl.pallas_call(
        _post_body,
        grid=(GRID,),
        in_specs=[
            pl.BlockSpec((NC, MBLK, D), lambda i: (0, i, 0)),
            pl.BlockSpec((MBLK, D), lambda i: (i, 0)),
            pl.BlockSpec((MBLK, 1), lambda i: (i, 0)),
            pl.BlockSpec((1, D), lambda i: (0, 0)),
        ],
        out_specs=pl.BlockSpec((MBLK, D), lambda i: (i, 0)),
        out_shape=jax.ShapeDtypeStruct((N, D), jnp.float32),
    )(agg, y1, dinv, b1)


def kernel(nodes_embeddings, edges, W0, b0, W1, b1):
    src = jnp.concatenate(
        [edges[0].astype(jnp.int32), jnp.zeros((EPAD - E,), jnp.int32)]
    ).reshape(NW, CHUNKS, CK)
    dst = jnp.concatenate(
        [edges[1].astype(jnp.int32), jnp.full((EPAD - E,), N, jnp.int32)]
    ).reshape(NW, CHUNKS, CK)

    zeros_deg = jnp.zeros((ZROWS, DEGW), jnp.float32)
    ones_deg = jnp.ones((CK, DEGW), jnp.float32)
    zeros_agg = jnp.zeros((ZROWS, D), jnp.float32)
    b0r = b0.reshape(1, D)
    b1r = b1.reshape(1, D)

    deg_kernel, agg_kernel = _sc_kernels()
    deg = deg_kernel(dst, zeros_deg, ones_deg)

    dinv, y0 = _pre_call(nodes_embeddings, W0, deg)
    agg0 = agg_kernel(y0, src, dst, zeros_agg)
    y1 = _mid_call(agg0, y0, dinv, W1, b0r)
    agg1 = agg_kernel(y1, src, dst, zeros_agg)
    return _post_call(agg1, y1, dinv, b1r)


# speedup vs baseline: 24.3588x; 2.5629x over previous
import functools

import jax
import jax.numpy as jnp
from jax import lax
from jax.experimental import pallas as pl
from jax.experimental.pallas import tpu as pltpu
from jax.experimental.pallas import tpu_sc as plsc

N = 10000
D = 128
E = 320000

NC = 2
NS = 16
NW = NC * NS

CK = 128
EPAD = 327680
EW = EPAD // NW
CHUNKS = EW // CK
NPAD = 10112
ZROWS = NPAD // NS
DEGW = 128
NBUF = 4
NIB = 4

MBLK = 1000
GRID = N // MBLK


@functools.lru_cache(maxsize=None)
def _sc_kernels():
    mesh = plsc.VectorSubcoreMesh(core_axis_name="c", subcore_axis_name="s",
                                  num_cores=NC, num_subcores=NS)

    @functools.partial(
        pl.kernel,
        out_type=jax.ShapeDtypeStruct((NC, NPAD, DEGW), jnp.float32),
        mesh=mesh,
        scratch_types=[
            pltpu.VMEM_SHARED((NPAD, DEGW), jnp.float32),
            pltpu.VMEM((CHUNKS, CK), jnp.int32),
            pltpu.VMEM((CK, DEGW), jnp.float32),
            pltpu.SemaphoreType.DMA,
        ],
    )
    def deg_kernel(dst_hbm, zeros_hbm, ones_hbm, out_hbm,
                   deg_sh, dst_buf, ones_v, sem):
        cid = lax.axis_index("c")
        sid = lax.axis_index("s")
        wid = cid * NS + sid
        pltpu.sync_copy(zeros_hbm, deg_sh.at[pl.ds(sid * ZROWS, ZROWS)])
        pltpu.sync_copy(ones_hbm, ones_v)
        pltpu.sync_copy(dst_hbm.at[wid], dst_buf)
        plsc.subcore_barrier()

        def dummy_wait(s):
            pltpu.make_async_copy(ones_hbm, ones_v, s).wait()

        def grp(g, carry):
            for b in range(NBUF):
                c = g * NBUF + b
                pltpu.async_copy(ones_v, deg_sh.at[dst_buf.at[c]], sem,
                                 add=True)

                @pl.when(g > 0)
                def _():
                    dummy_wait(sem)
            return carry

        lax.fori_loop(0, CHUNKS // NBUF, grp, 0)
        for b in range(NBUF):
            dummy_wait(sem)
        plsc.subcore_barrier()
        pltpu.sync_copy(deg_sh.at[pl.ds(sid * ZROWS, ZROWS)],
                        out_hbm.at[cid, pl.ds(sid * ZROWS, ZROWS)])

    @functools.partial(
        pl.kernel,
        out_type=jax.ShapeDtypeStruct((NC, NPAD, D), jnp.float32),
        mesh=mesh,
        scratch_types=[
            pltpu.VMEM_SHARED((NPAD, D), jnp.float32),
            pltpu.VMEM((NIB, CK), jnp.int32),
            pltpu.VMEM((NIB, CK), jnp.int32),
            pltpu.VMEM((2, CK, D), jnp.float32),
            pltpu.SemaphoreType.DMA((NIB,)),
            pltpu.SemaphoreType.DMA((2,)),
            pltpu.SemaphoreType.DMA((2,)),
        ],
    )
    def agg_kernel(y_hbm, src_hbm, dst_hbm, zeros_hbm, out_hbm,
                   agg_sh, src_i, dst_i, rows_v, isem, gsem, ssem):
        cid = lax.axis_index("c")
        sid = lax.axis_index("s")
        wid = cid * NS + sid
        pltpu.sync_copy(zeros_hbm, agg_sh.at[pl.ds(sid * ZROWS, ZROWS)])
        plsc.subcore_barrier()

        def fire_idx(c, islot):
            pltpu.async_copy(src_hbm.at[wid, c], src_i.at[islot],
                             isem.at[islot])
            pltpu.async_copy(dst_hbm.at[wid, c], dst_i.at[islot],
                             isem.at[islot])

        def wait_idx(islot):
            for _ in range(2):
                pltpu.make_async_copy(src_hbm.at[0, 0], src_i.at[islot],
                                      isem.at[islot]).wait()

        def fire_gather(c_islot, rslot):
            pltpu.async_copy(y_hbm.at[src_i.at[c_islot]], rows_v.at[rslot],
                             gsem.at[rslot])

        def wait_rows(rslot, s):
            pltpu.make_async_copy(y_hbm.at[pl.ds(0, CK)], rows_v.at[rslot],
                                  s.at[rslot]).wait()

        for c0 in range(3):
            fire_idx(c0, c0)
        wait_idx(0)
        fire_gather(0, 0)

        def grp(g, carry):
            for b in range(4):
                c = g * 4 + b
                rslot = b % 2
                oslot = 1 - rslot
                wait_rows(rslot, gsem)
                pltpu.async_copy(rows_v.at[rslot],
                                 agg_sh.at[dst_i.at[b]],
                                 ssem.at[rslot], add=True)

                @pl.when(c >= 1)
                def _():
                    wait_rows(oslot, ssem)

                @pl.when(c + 3 < CHUNKS)
                def _():
                    fire_idx(c + 3, (b + 3) % NIB)

                @pl.when(c + 1 < CHUNKS)
                def _():
                    wait_idx((b + 1) % NIB)
                    fire_gather((b + 1) % NIB, oslot)
            return carry

        lax.fori_loop(0, CHUNKS // 4, grp, 0)
        wait_rows((CHUNKS - 1) % 2, ssem)
        plsc.subcore_barrier()
        pltpu.sync_copy(agg_sh.at[pl.ds(sid * ZROWS, ZROWS)],
                        out_hbm.at[cid, pl.ds(sid * ZROWS, ZROWS)])

    return deg_kernel, agg_kernel


def _pre_body(x_ref, w_ref, deg_ref, dinv_ref, y_ref):
    deg = deg_ref[0, :, :1] + deg_ref[1, :, :1] + 1.0
    dinv = lax.rsqrt(deg)
    dinv_ref[...] = dinv
    y_ref[...] = dinv * jnp.dot(x_ref[...], w_ref[...],
                                preferred_element_type=jnp.float32)


def _pre_call(x, w0, deg):
    return pl.pallas_call(
        _pre_body,
        grid=(GRID,),
        in_specs=[
            pl.BlockSpec((MBLK, D), lambda i: (i, 0)),
            pl.BlockSpec((D, D), lambda i: (0, 0)),
            pl.BlockSpec((NC, MBLK, DEGW), lambda i: (0, i, 0)),
        ],
        out_specs=[
            pl.BlockSpec((MBLK, 1), lambda i: (i, 0)),
            pl.BlockSpec((MBLK, D), lambda i: (i, 0)),
        ],
        out_shape=[
            jax.ShapeDtypeStruct((N, 1), jnp.float32),
            jax.ShapeDtypeStruct((N, D), jnp.float32),
        ],
    )(x, w0, deg)


def _mid_body(agg_ref, y_ref, dinv_ref, w_ref, b_ref, y1_ref):
    dinv = dinv_ref[...]
    h = dinv * (agg_ref[0] + agg_ref[1] + y_ref[...]) + b_ref[...]
    y1_ref[...] = dinv * jnp.dot(h, w_ref[...],
                                 preferred_element_type=jnp.float32)


def _mid_call(agg, y0, dinv, w1, b0):
    return pl.pallas_call(
        _mid_body,
        grid=(GRID,),
        in_specs=[
            pl.BlockSpec((NC, MBLK, D), lambda i: (0, i, 0)),
            pl.BlockSpec((MBLK, D), lambda i: (i, 0)),
            pl.BlockSpec((MBLK, 1), lambda i: (i, 0)),
            pl.BlockSpec((D, D), lambda i: (0, 0)),
            pl.BlockSpec((1, D), lambda i: (0, 0)),
        ],
        out_specs=pl.BlockSpec((MBLK, D), lambda i: (i, 0)),
        out_shape=jax.ShapeDtypeStruct((N, D), jnp.float32),
    )(agg, y0, dinv, w1, b0)


def _post_body(agg_ref, y_ref, dinv_ref, b_ref, out_ref):
    out_ref[...] = (dinv_ref[...] * (agg_ref[0] + agg_ref[1] + y_ref[...])
                    + b_ref[...])


def _post_call(agg, y1, dinv, b1):
    return pl.pallas_call(
        _post_body,
        grid=(GRID,),
        in_specs=[
            pl.BlockSpec((NC, MBLK, D), lambda i: (0, i, 0)),
            pl.BlockSpec((MBLK, D), lambda i: (i, 0)),
            pl.BlockSpec((MBLK, 1), lambda i: (i, 0)),
            pl.BlockSpec((1, D), lambda i: (0, 0)),
        ],
        out_specs=pl.BlockSpec((MBLK, D), lambda i: (i, 0)),
        out_shape=jax.ShapeDtypeStruct((N, D), jnp.float32),
    )(agg, y1, dinv, b1)


def kernel(nodes_embeddings, edges, W0, b0, W1, b1):
    npad_e = EPAD - E
    pad_src = (jnp.arange(npad_e, dtype=jnp.int32) * 97) % N
    pad_dst = N + (jnp.arange(npad_e, dtype=jnp.int32) % (NPAD - N))
    src = jnp.concatenate(
        [edges[0].astype(jnp.int32), pad_src]).reshape(NW, CHUNKS, CK)
    dst = jnp.concatenate(
        [edges[1].astype(jnp.int32), pad_dst]).reshape(NW, CHUNKS, CK)

    zeros_deg = jnp.zeros((ZROWS, DEGW), jnp.float32)
    ones_deg = jnp.ones((CK, DEGW), jnp.float32)
    zeros_agg = jnp.zeros((ZROWS, D), jnp.float32)
    b0r = b0.reshape(1, D)
    b1r = b1.reshape(1, D)

    deg_kernel, agg_kernel = _sc_kernels()
    deg = deg_kernel(dst, zeros_deg, ones_deg)

    dinv, y0 = _pre_call(nodes_embeddings, W0, deg)
    agg0 = agg_kernel(y0, src, dst, zeros_agg)
    y1 = _mid_call(agg0, y0, dinv, W1, b0r)
    agg1 = agg_kernel(y1, src, dst, zeros_agg)
    return _post_call(agg1, y1, dinv, b1r)


# speedup vs baseline: 25.2931x; 1.0384x over previous
import functools

import jax
import jax.numpy as jnp
from jax import lax
from jax.experimental import pallas as pl
from jax.experimental.pallas import tpu as pltpu
from jax.experimental.pallas import tpu_sc as plsc

N = 10000
D = 128
E = 320000

NC = 2
NS = 16
NW = NC * NS

CK = 64
EPAD = 327680
EW = EPAD // NW
CHUNKS = EW // CK
NPAD = 10112
ZROWS = NPAD // NS
DEGW = 128
NBUF = 4
NIB = 8
RB = 4

MBLK = 1000
GRID = N // MBLK


@functools.lru_cache(maxsize=None)
def _sc_kernels():
    mesh = plsc.VectorSubcoreMesh(core_axis_name="c", subcore_axis_name="s",
                                  num_cores=NC, num_subcores=NS)

    @functools.partial(
        pl.kernel,
        out_type=jax.ShapeDtypeStruct((NC, NPAD, DEGW), jnp.float32),
        mesh=mesh,
        scratch_types=[
            pltpu.VMEM_SHARED((NPAD, DEGW), jnp.float32),
            pltpu.VMEM((CHUNKS, CK), jnp.int32),
            pltpu.VMEM((CK, DEGW), jnp.float32),
            pltpu.SemaphoreType.DMA,
        ],
    )
    def deg_kernel(dst_hbm, zeros_hbm, ones_hbm, out_hbm,
                   deg_sh, dst_buf, ones_v, sem):
        cid = lax.axis_index("c")
        sid = lax.axis_index("s")
        wid = cid * NS + sid
        pltpu.sync_copy(zeros_hbm, deg_sh.at[pl.ds(sid * ZROWS, ZROWS)])
        pltpu.sync_copy(ones_hbm, ones_v)
        pltpu.sync_copy(dst_hbm.at[wid], dst_buf)
        plsc.subcore_barrier()

        def dummy_wait(s):
            pltpu.make_async_copy(ones_hbm, ones_v, s).wait()

        def grp(g, carry):
            for b in range(NBUF):
                c = g * NBUF + b
                pltpu.async_copy(ones_v, deg_sh.at[dst_buf.at[c]], sem,
                                 add=True)

                @pl.when(g > 0)
                def _():
                    dummy_wait(sem)
            return carry

        lax.fori_loop(0, CHUNKS // NBUF, grp, 0)
        for b in range(NBUF):
            dummy_wait(sem)
        plsc.subcore_barrier()
        pltpu.sync_copy(deg_sh.at[pl.ds(sid * ZROWS, ZROWS)],
                        out_hbm.at[cid, pl.ds(sid * ZROWS, ZROWS)])

    @functools.partial(
        pl.kernel,
        out_type=jax.ShapeDtypeStruct((NC, NPAD, D), jnp.float32),
        mesh=mesh,
        scratch_types=[
            pltpu.VMEM_SHARED((NPAD, D), jnp.float32),
            pltpu.VMEM((NIB, CK), jnp.int32),
            pltpu.VMEM((NIB, CK), jnp.int32),
            pltpu.VMEM((RB, CK, D), jnp.float32),
            pltpu.SemaphoreType.DMA((NIB,)),
            pltpu.SemaphoreType.DMA((RB,)),
            pltpu.SemaphoreType.DMA((RB,)),
        ],
    )
    def agg_kernel(y_hbm, src_hbm, dst_hbm, zeros_hbm, out_hbm,
                   agg_sh, src_i, dst_i, rows_v, isem, gsem, ssem):
        cid = lax.axis_index("c")
        sid = lax.axis_index("s")
        wid = cid * NS + sid
        pltpu.sync_copy(zeros_hbm, agg_sh.at[pl.ds(sid * ZROWS, ZROWS)])
        plsc.subcore_barrier()

        def fire_idx(c, islot):
            pltpu.async_copy(src_hbm.at[wid, c], src_i.at[islot],
                             isem.at[islot])
            pltpu.async_copy(dst_hbm.at[wid, c], dst_i.at[islot],
                             isem.at[islot])

        def wait_idx(islot):
            for _ in range(2):
                pltpu.make_async_copy(src_hbm.at[0, 0], src_i.at[islot],
                                      isem.at[islot]).wait()

        def fire_gather(c_islot, rslot):
            pltpu.async_copy(y_hbm.at[src_i.at[c_islot]], rows_v.at[rslot],
                             gsem.at[rslot])

        def wait_rows(rslot, s):
            pltpu.make_async_copy(y_hbm.at[pl.ds(0, CK)], rows_v.at[rslot],
                                  s.at[rslot]).wait()

        for c0 in range(4):
            fire_idx(c0, c0)
        wait_idx(0)
        fire_gather(0, 0)
        wait_idx(1)
        fire_gather(1, 1)

        def grp(g, carry):
            for b in range(8):
                c = g * 8 + b
                rslot = b % RB
                nslot = (b + 2) % RB
                wait_rows(rslot, gsem)
                pltpu.async_copy(rows_v.at[rslot],
                                 agg_sh.at[dst_i.at[b]],
                                 ssem.at[rslot], add=True)

                @pl.when(c >= 2)
                def _():
                    wait_rows(nslot, ssem)

                @pl.when(c + 4 < CHUNKS)
                def _():
                    fire_idx(c + 4, (b + 4) % NIB)

                @pl.when(c + 2 < CHUNKS)
                def _():
                    wait_idx((b + 2) % NIB)
                    fire_gather((b + 2) % NIB, nslot)
            return carry

        lax.fori_loop(0, CHUNKS // 8, grp, 0)
        wait_rows((CHUNKS - 2) % RB, ssem)
        wait_rows((CHUNKS - 1) % RB, ssem)
        plsc.subcore_barrier()
        pltpu.sync_copy(agg_sh.at[pl.ds(sid * ZROWS, ZROWS)],
                        out_hbm.at[cid, pl.ds(sid * ZROWS, ZROWS)])

    return deg_kernel, agg_kernel


def _pre_body(x_ref, w_ref, deg_ref, dinv_ref, y_ref):
    deg = deg_ref[0, :, :1] + deg_ref[1, :, :1] + 1.0
    dinv = lax.rsqrt(deg)
    dinv_ref[...] = dinv
    y_ref[...] = dinv * jnp.dot(x_ref[...], w_ref[...],
                                preferred_element_type=jnp.float32)


def _pre_call(x, w0, deg):
    return pl.pallas_call(
        _pre_body,
        grid=(GRID,),
        in_specs=[
            pl.BlockSpec((MBLK, D), lambda i: (i, 0)),
            pl.BlockSpec((D, D), lambda i: (0, 0)),
            pl.BlockSpec((NC, MBLK, DEGW), lambda i: (0, i, 0)),
        ],
        out_specs=[
            pl.BlockSpec((MBLK, 1), lambda i: (i, 0)),
            pl.BlockSpec((MBLK, D), lambda i: (i, 0)),
        ],
        out_shape=[
            jax.ShapeDtypeStruct((N, 1), jnp.float32),
            jax.ShapeDtypeStruct((N, D), jnp.float32),
        ],
    )(x, w0, deg)


def _mid_body(agg_ref, y_ref, dinv_ref, w_ref, b_ref, y1_ref):
    dinv = dinv_ref[...]
    h = dinv * (agg_ref[0] + agg_ref[1] + y_ref[...]) + b_ref[...]
    y1_ref[...] = dinv * jnp.dot(h, w_ref[...],
                                 preferred_element_type=jnp.float32)


def _mid_call(agg, y0, dinv, w1, b0):
    return pl.pallas_call(
        _mid_body,
        grid=(GRID,),
        in_specs=[
            pl.BlockSpec((NC, MBLK, D), lambda i: (0, i, 0)),
            pl.BlockSpec((MBLK, D), lambda i: (i, 0)),
            pl.BlockSpec((MBLK, 1), lambda i: (i, 0)),
            pl.BlockSpec((D, D), lambda i: (0, 0)),
            pl.BlockSpec((1, D), lambda i: (0, 0)),
        ],
        out_specs=pl.BlockSpec((MBLK, D), lambda i: (i, 0)),
        out_shape=jax.ShapeDtypeStruct((N, D), jnp.float32),
    )(agg, y0, dinv, w1, b0)


def _post_body(agg_ref, y_ref, dinv_ref, b_ref, out_ref):
    out_ref[...] = (dinv_ref[...] * (agg_ref[0] + agg_ref[1] + y_ref[...])
                    + b_ref[...])


def _post_call(agg, y1, dinv, b1):
    return pl.pallas_call(
        _post_body,
        grid=(GRID,),
        in_specs=[
            pl.BlockSpec((NC, MBLK, D), lambda i: (0, i, 0)),
            pl.BlockSpec((MBLK, D), lambda i: (i, 0)),
            pl.BlockSpec((MBLK, 1), lambda i: (i, 0)),
            pl.BlockSpec((1, D), lambda i: (0, 0)),
        ],
        out_specs=pl.BlockSpec((MBLK, D), lambda i: (i, 0)),
        out_shape=jax.ShapeDtypeStruct((N, D), jnp.float32),
    )(agg, y1, dinv, b1)


def kernel(nodes_embeddings, edges, W0, b0, W1, b1):
    npad_e = EPAD - E
    pad_src = (jnp.arange(npad_e, dtype=jnp.int32) * 97) % N
    pad_dst = N + (jnp.arange(npad_e, dtype=jnp.int32) % (NPAD - N))
    src = jnp.concatenate(
        [edges[0].astype(jnp.int32), pad_src]).reshape(NW, CHUNKS, CK)
    dst = jnp.concatenate(
        [edges[1].astype(jnp.int32), pad_dst]).reshape(NW, CHUNKS, CK)

    zeros_deg = jnp.zeros((ZROWS, DEGW), jnp.float32)
    ones_deg = jnp.ones((CK, DEGW), jnp.float32)
    zeros_agg = jnp.zeros((ZROWS, D), jnp.float32)
    b0r = b0.reshape(1, D)
    b1r = b1.reshape(1, D)

    deg_kernel, agg_kernel = _sc_kernels()
    deg = deg_kernel(dst, zeros_deg, ones_deg)

    dinv, y0 = _pre_call(nodes_embeddings, W0, deg)
    agg0 = agg_kernel(y0, src, dst, zeros_agg)
    y1 = _mid_call(agg0, y0, dinv, W1, b0r)
    agg1 = agg_kernel(y1, src, dst, zeros_agg)
    return _post_call(agg1, y1, dinv, b1r)


# speedup vs baseline: 25.4635x; 1.0067x over previous
import functools

import jax
import jax.numpy as jnp
from jax import lax
from jax.experimental import pallas as pl
from jax.experimental.pallas import tpu as pltpu
from jax.experimental.pallas import tpu_sc as plsc

N = 10000
D = 128
E = 320000

NC = 2
NS = 16
NW = NC * NS

CK = 64
EPAD = 327680
EW = EPAD // NW
CHUNKS = EW // CK
NPAD = 10112
ZROWS = NPAD // NS
DEGW = 128
NBUF = 4
NIB = 8
RB = 4

MBLK = 1000
GRID = N // MBLK


@functools.lru_cache(maxsize=None)
def _sc_kernels():
    mesh = plsc.VectorSubcoreMesh(core_axis_name="c", subcore_axis_name="s",
                                  num_cores=NC, num_subcores=NS)

    @functools.partial(
        pl.kernel,
        out_type=jax.ShapeDtypeStruct((NC, NPAD, DEGW), jnp.float32),
        mesh=mesh,
        scratch_types=[
            pltpu.VMEM_SHARED((NPAD, DEGW), jnp.float32),
            pltpu.VMEM((CHUNKS, CK), jnp.int32),
            pltpu.VMEM((CK, DEGW), jnp.float32),
            pltpu.SemaphoreType.DMA,
        ],
    )
    def deg_kernel(dst_hbm, zeros_hbm, ones_hbm, out_hbm,
                   deg_sh, dst_buf, ones_v, sem):
        cid = lax.axis_index("c")
        sid = lax.axis_index("s")
        wid = cid * NS + sid
        pltpu.async_copy(dst_hbm.at[wid], dst_buf, sem)
        pltpu.async_copy(ones_hbm, ones_v, sem)
        pltpu.sync_copy(zeros_hbm, deg_sh.at[pl.ds(sid * ZROWS, ZROWS)])
        pltpu.make_async_copy(dst_hbm.at[wid], dst_buf, sem).wait()
        pltpu.make_async_copy(ones_hbm, ones_v, sem).wait()
        plsc.subcore_barrier()

        def dummy_wait(s):
            pltpu.make_async_copy(ones_hbm, ones_v, s).wait()

        def grp(g, carry):
            for b in range(NBUF):
                c = g * NBUF + b
                pltpu.async_copy(ones_v, deg_sh.at[dst_buf.at[c]], sem,
                                 add=True)

                @pl.when(g > 0)
                def _():
                    dummy_wait(sem)
            return carry

        lax.fori_loop(0, CHUNKS // NBUF, grp, 0)
        for b in range(NBUF):
            dummy_wait(sem)
        plsc.subcore_barrier()
        pltpu.sync_copy(deg_sh.at[pl.ds(sid * ZROWS, ZROWS)],
                        out_hbm.at[cid, pl.ds(sid * ZROWS, ZROWS)])

    @functools.partial(
        pl.kernel,
        out_type=jax.ShapeDtypeStruct((NC, NPAD, D), jnp.float32),
        mesh=mesh,
        scratch_types=[
            pltpu.VMEM_SHARED((NPAD, D), jnp.float32),
            pltpu.VMEM((NIB, CK), jnp.int32),
            pltpu.VMEM((NIB, CK), jnp.int32),
            pltpu.VMEM((RB, CK, D), jnp.float32),
            pltpu.SemaphoreType.DMA((NIB,)),
            pltpu.SemaphoreType.DMA((RB,)),
            pltpu.SemaphoreType.DMA((RB,)),
        ],
    )
    def agg_kernel(y_hbm, src_hbm, dst_hbm, zeros_hbm, out_hbm,
                   agg_sh, src_i, dst_i, rows_v, isem, gsem, ssem):
        cid = lax.axis_index("c")
        sid = lax.axis_index("s")
        wid = cid * NS + sid

        def fire_idx(c, islot):
            pltpu.async_copy(src_hbm.at[wid, c], src_i.at[islot],
                             isem.at[islot])
            pltpu.async_copy(dst_hbm.at[wid, c], dst_i.at[islot],
                             isem.at[islot])

        def wait_idx(islot):
            for _ in range(2):
                pltpu.make_async_copy(src_hbm.at[0, 0], src_i.at[islot],
                                      isem.at[islot]).wait()

        def fire_gather(c_islot, rslot):
            pltpu.async_copy(y_hbm.at[src_i.at[c_islot]], rows_v.at[rslot],
                             gsem.at[rslot])

        def wait_rows(rslot, s):
            pltpu.make_async_copy(y_hbm.at[pl.ds(0, CK)], rows_v.at[rslot],
                                  s.at[rslot]).wait()

        for c0 in range(4):
            fire_idx(c0, c0)
        wait_idx(0)
        fire_gather(0, 0)
        wait_idx(1)
        fire_gather(1, 1)
        pltpu.sync_copy(zeros_hbm, agg_sh.at[pl.ds(sid * ZROWS, ZROWS)])
        plsc.subcore_barrier()

        def grp(g, carry):
            for b in range(8):
                c = g * 8 + b
                rslot = b % RB
                nslot = (b + 2) % RB
                wait_rows(rslot, gsem)
                pltpu.async_copy(rows_v.at[rslot],
                                 agg_sh.at[dst_i.at[b]],
                                 ssem.at[rslot], add=True)

                @pl.when(c >= 2)
                def _():
                    wait_rows(nslot, ssem)

                @pl.when(c + 4 < CHUNKS)
                def _():
                    fire_idx(c + 4, (b + 4) % NIB)

                @pl.when(c + 2 < CHUNKS)
                def _():
                    wait_idx((b + 2) % NIB)
                    fire_gather((b + 2) % NIB, nslot)
            return carry

        lax.fori_loop(0, CHUNKS // 8, grp, 0)
        wait_rows((CHUNKS - 2) % RB, ssem)
        wait_rows((CHUNKS - 1) % RB, ssem)
        plsc.subcore_barrier()
        pltpu.sync_copy(agg_sh.at[pl.ds(sid * ZROWS, ZROWS)],
                        out_hbm.at[cid, pl.ds(sid * ZROWS, ZROWS)])

    return deg_kernel, agg_kernel


def _mm_body(x_ref, w_ref, o_ref):
    o_ref[...] = jnp.dot(x_ref[...], w_ref[...],
                         preferred_element_type=jnp.float32)


def _mm_call(x, w0):
    return pl.pallas_call(
        _mm_body,
        grid=(GRID,),
        in_specs=[
            pl.BlockSpec((MBLK, D), lambda i: (i, 0)),
            pl.BlockSpec((D, D), lambda i: (0, 0)),
        ],
        out_specs=pl.BlockSpec((MBLK, D), lambda i: (i, 0)),
        out_shape=jax.ShapeDtypeStruct((N, D), jnp.float32),
    )(x, w0)


def _scale_body(deg_ref, xw_ref, dinv_ref, y_ref):
    deg = deg_ref[0, :, :1] + deg_ref[1, :, :1] + 1.0
    dinv = lax.rsqrt(deg)
    dinv_ref[...] = dinv
    y_ref[...] = dinv * xw_ref[...]


def _scale_call(deg, xw):
    return pl.pallas_call(
        _scale_body,
        grid=(GRID,),
        in_specs=[
            pl.BlockSpec((NC, MBLK, DEGW), lambda i: (0, i, 0)),
            pl.BlockSpec((MBLK, D), lambda i: (i, 0)),
        ],
        out_specs=[
            pl.BlockSpec((MBLK, 1), lambda i: (i, 0)),
            pl.BlockSpec((MBLK, D), lambda i: (i, 0)),
        ],
        out_shape=[
            jax.ShapeDtypeStruct((N, 1), jnp.float32),
            jax.ShapeDtypeStruct((N, D), jnp.float32),
        ],
    )(deg, xw)


def _mid_body(agg_ref, y_ref, dinv_ref, w_ref, b_ref, y1_ref):
    dinv = dinv_ref[...]
    h = dinv * (agg_ref[0] + agg_ref[1] + y_ref[...]) + b_ref[...]
    y1_ref[...] = dinv * jnp.dot(h, w_ref[...],
                                 preferred_element_type=jnp.float32)


def _mid_call(agg, y0, dinv, w1, b0):
    return pl.pallas_call(
        _mid_body,
        grid=(GRID,),
        in_specs=[
            pl.BlockSpec((NC, MBLK, D), lambda i: (0, i, 0)),
            pl.BlockSpec((MBLK, D), lambda i: (i, 0)),
            pl.BlockSpec((MBLK, 1), lambda i: (i, 0)),
            pl.BlockSpec((D, D), lambda i: (0, 0)),
            pl.BlockSpec((1, D), lambda i: (0, 0)),
        ],
        out_specs=pl.BlockSpec((MBLK, D), lambda i: (i, 0)),
        out_shape=jax.ShapeDtypeStruct((N, D), jnp.float32),
    )(agg, y0, dinv, w1, b0)


def _post_body(agg_ref, y_ref, dinv_ref, b_ref, out_ref):
    out_ref[...] = (dinv_ref[...] * (agg_ref[0] + agg_ref[1] + y_ref[...])
                    + b_ref[...])


def _post_call(agg, y1, dinv, b1):
    return pl.pallas_call(
        _post_body,
        grid=(GRID,),
        in_specs=[
            pl.BlockSpec((NC, MBLK, D), lambda i: (0, i, 0)),
            pl.BlockSpec((MBLK, D), lambda i: (i, 0)),
            pl.BlockSpec((MBLK, 1), lambda i: (i, 0)),
            pl.BlockSpec((1, D), lambda i: (0, 0)),
        ],
        out_specs=pl.BlockSpec((MBLK, D), lambda i: (i, 0)),
        out_shape=jax.ShapeDtypeStruct((N, D), jnp.float32),
    )(agg, y1, dinv, b1)


def kernel(nodes_embeddings, edges, W0, b0, W1, b1):
    npad_e = EPAD - E
    pad_src = (jnp.arange(npad_e, dtype=jnp.int32) * 97) % N
    pad_dst = N + (jnp.arange(npad_e, dtype=jnp.int32) % (NPAD - N))
    src = jnp.concatenate(
        [edges[0].astype(jnp.int32), pad_src]).reshape(NW, CHUNKS, CK)
    dst = jnp.concatenate(
        [edges[1].astype(jnp.int32), pad_dst]).reshape(NW, CHUNKS, CK)

    zeros_deg = jnp.zeros((ZROWS, DEGW), jnp.float32)
    ones_deg = jnp.ones((CK, DEGW), jnp.float32)
    zeros_agg = jnp.zeros((ZROWS, D), jnp.float32)
    b0r = b0.reshape(1, D)
    b1r = b1.reshape(1, D)

    deg_kernel, agg_kernel = _sc_kernels()
    deg = deg_kernel(dst, zeros_deg, ones_deg)

    xw0 = _mm_call(nodes_embeddings, W0)
    dinv, y0 = _scale_call(deg, xw0)
    agg0 = agg_kernel(y0, src, dst, zeros_agg)
    y1 = _mid_call(agg0, y0, dinv, W1, b0r)
    agg1 = agg_kernel(y1, src, dst, zeros_agg)
    return _post_call(agg1, y1, dinv, b1r)


# speedup vs baseline: 28.4649x; 1.1179x over previous
import functools

import jax
import jax.numpy as jnp
from jax import lax
from jax.experimental import pallas as pl
from jax.experimental.pallas import tpu as pltpu
from jax.experimental.pallas import tpu_sc as plsc

N = 10000
D = 128
E = 320000

NC = 2
NS = 16
NW = NC * NS

CK = 64
EPAD = 327680
EW = EPAD // NW
CHUNKS = EW // CK
NPAD = 10112
ZROWS = NPAD // NS
DEGW = 128
NBUF = 4
NIB = 10
RB = 5

MBLK = 1000
GRID = N // MBLK


@functools.lru_cache(maxsize=None)
def _sc_kernels():
    mesh = plsc.VectorSubcoreMesh(core_axis_name="c", subcore_axis_name="s",
                                  num_cores=NC, num_subcores=NS)

    @functools.partial(
        pl.kernel,
        out_type=jax.ShapeDtypeStruct((NC, NPAD, DEGW), jnp.float32),
        mesh=mesh,
        scratch_types=[
            pltpu.VMEM_SHARED((NPAD, DEGW), jnp.float32),
            pltpu.VMEM((CHUNKS, CK), jnp.int32),
            pltpu.VMEM((CK, DEGW), jnp.float32),
            pltpu.SemaphoreType.DMA,
        ],
    )
    def deg_kernel(dst_hbm, zeros_hbm, ones_hbm, out_hbm,
                   deg_sh, dst_buf, ones_v, sem):
        cid = lax.axis_index("c")
        sid = lax.axis_index("s")
        wid = cid * NS + sid
        pltpu.async_copy(dst_hbm.at[wid], dst_buf, sem)
        pltpu.async_copy(ones_hbm, ones_v, sem)
        pltpu.sync_copy(zeros_hbm, deg_sh.at[pl.ds(sid * ZROWS, ZROWS)])
        pltpu.make_async_copy(dst_hbm.at[wid], dst_buf, sem).wait()
        pltpu.make_async_copy(ones_hbm, ones_v, sem).wait()
        plsc.subcore_barrier()

        def dummy_wait(s):
            pltpu.make_async_copy(ones_hbm, ones_v, s).wait()

        def grp(g, carry):
            for b in range(NBUF):
                c = g * NBUF + b
                pltpu.async_copy(ones_v, deg_sh.at[dst_buf.at[c]], sem,
                                 add=True)

                @pl.when(g > 0)
                def _():
                    dummy_wait(sem)
            return carry

        lax.fori_loop(0, CHUNKS // NBUF, grp, 0)
        for b in range(NBUF):
            dummy_wait(sem)
        plsc.subcore_barrier()
        pltpu.sync_copy(deg_sh.at[pl.ds(sid * ZROWS, ZROWS)],
                        out_hbm.at[cid, pl.ds(sid * ZROWS, ZROWS)])

    @functools.partial(
        pl.kernel,
        out_type=jax.ShapeDtypeStruct((NC, NPAD, D), jnp.float32),
        mesh=mesh,
        scratch_types=[
            pltpu.VMEM_SHARED((NPAD, D), jnp.float32),
            pltpu.VMEM((NIB, CK), jnp.int32),
            pltpu.VMEM((NIB, CK), jnp.int32),
            pltpu.VMEM((RB, CK, D), jnp.float32),
            pltpu.SemaphoreType.DMA((NIB,)),
            pltpu.SemaphoreType.DMA((RB,)),
            pltpu.SemaphoreType.DMA((RB,)),
        ],
    )
    def agg_kernel(y_hbm, src_hbm, dst_hbm, zeros_hbm, out_hbm,
                   agg_sh, src_i, dst_i, rows_v, isem, gsem, ssem):
        cid = lax.axis_index("c")
        sid = lax.axis_index("s")
        wid = cid * NS + sid

        def fire_idx(c, islot):
            pltpu.async_copy(src_hbm.at[wid, c], src_i.at[islot],
                             isem.at[islot])
            pltpu.async_copy(dst_hbm.at[wid, c], dst_i.at[islot],
                             isem.at[islot])

        def wait_idx(islot):
            for _ in range(2):
                pltpu.make_async_copy(src_hbm.at[0, 0], src_i.at[islot],
                                      isem.at[islot]).wait()

        def fire_gather(c_islot, rslot):
            pltpu.async_copy(y_hbm.at[src_i.at[c_islot]], rows_v.at[rslot],
                             gsem.at[rslot])

        def wait_rows(rslot, s):
            pltpu.make_async_copy(y_hbm.at[pl.ds(0, CK)], rows_v.at[rslot],
                                  s.at[rslot]).wait()

        for c0 in range(5):
            fire_idx(c0, c0)
        for c0 in range(3):
            wait_idx(c0)
            fire_gather(c0, c0)
        pltpu.sync_copy(zeros_hbm, agg_sh.at[pl.ds(sid * ZROWS, ZROWS)])
        plsc.subcore_barrier()

        def grp(g, carry):
            for b in range(10):
                c = g * 10 + b
                rslot = b % RB
                nslot = (b + 3) % RB
                wait_rows(rslot, gsem)
                pltpu.async_copy(rows_v.at[rslot],
                                 agg_sh.at[dst_i.at[b % NIB]],
                                 ssem.at[rslot], add=True)

                @pl.when(c >= 2)
                def _():
                    wait_rows(nslot, ssem)

                @pl.when(c + 5 < CHUNKS)
                def _():
                    fire_idx(c + 5, (b + 5) % NIB)

                @pl.when(c + 3 < CHUNKS)
                def _():
                    wait_idx((b + 3) % NIB)
                    fire_gather((b + 3) % NIB, nslot)
            return carry

        lax.fori_loop(0, CHUNKS // 10, grp, 0)
        wait_rows((CHUNKS - 2) % RB, ssem)
        wait_rows((CHUNKS - 1) % RB, ssem)
        plsc.subcore_barrier()
        pltpu.sync_copy(agg_sh.at[pl.ds(sid * ZROWS, ZROWS)],
                        out_hbm.at[cid, pl.ds(sid * ZROWS, ZROWS)])

    return deg_kernel, agg_kernel


def _mm_body(x_ref, w_ref, o_ref):
    o_ref[...] = jnp.dot(x_ref[...], w_ref[...],
                         preferred_element_type=jnp.float32)


def _mm_call(x, w0):
    return pl.pallas_call(
        _mm_body,
        grid=(GRID,),
        in_specs=[
            pl.BlockSpec((MBLK, D), lambda i: (i, 0)),
            pl.BlockSpec((D, D), lambda i: (0, 0)),
        ],
        out_specs=pl.BlockSpec((MBLK, D), lambda i: (i, 0)),
        out_shape=jax.ShapeDtypeStruct((N, D), jnp.float32),
    )(x, w0)


def _scale_body(deg_ref, xw_ref, dinv_ref, y_ref):
    deg = deg_ref[0, :, :1] + deg_ref[1, :, :1] + 1.0
    dinv = lax.rsqrt(deg)
    dinv_ref[...] = dinv
    y_ref[...] = dinv * xw_ref[...]


def _scale_call(deg, xw):
    return pl.pallas_call(
        _scale_body,
        grid=(GRID,),
        in_specs=[
            pl.BlockSpec((NC, MBLK, DEGW), lambda i: (0, i, 0)),
            pl.BlockSpec((MBLK, D), lambda i: (i, 0)),
        ],
        out_specs=[
            pl.BlockSpec((MBLK, 1), lambda i: (i, 0)),
            pl.BlockSpec((MBLK, D), lambda i: (i, 0)),
        ],
        out_shape=[
            jax.ShapeDtypeStruct((N, 1), jnp.float32),
            jax.ShapeDtypeStruct((N, D), jnp.float32),
        ],
    )(deg, xw)


def _mid_body(agg_ref, y_ref, dinv_ref, w_ref, b_ref, y1_ref):
    dinv = dinv_ref[...]
    h = dinv * (agg_ref[0] + agg_ref[1] + y_ref[...]) + b_ref[...]
    y1_ref[...] = dinv * jnp.dot(h, w_ref[...],
                                 preferred_element_type=jnp.float32)


def _mid_call(agg, y0, dinv, w1, b0):
    return pl.pallas_call(
        _mid_body,
        grid=(GRID,),
        in_specs=[
            pl.BlockSpec((NC, MBLK, D), lambda i: (0, i, 0)),
            pl.BlockSpec((MBLK, D), lambda i: (i, 0)),
            pl.BlockSpec((MBLK, 1), lambda i: (i, 0)),
            pl.BlockSpec((D, D), lambda i: (0, 0)),
            pl.BlockSpec((1, D), lambda i: (0, 0)),
        ],
        out_specs=pl.BlockSpec((MBLK, D), lambda i: (i, 0)),
        out_shape=jax.ShapeDtypeStruct((N, D), jnp.float32),
    )(agg, y0, dinv, w1, b0)


def _post_body(agg_ref, y_ref, dinv_ref, b_ref, out_ref):
    out_ref[...] = (dinv_ref[...] * (agg_ref[0] + agg_ref[1] + y_ref[...])
                    + b_ref[...])


def _post_call(agg, y1, dinv, b1):
    return pl.pallas_call(
        _post_body,
        grid=(GRID,),
        in_specs=[
            pl.BlockSpec((NC, MBLK, D), lambda i: (0, i, 0)),
            pl.BlockSpec((MBLK, D), lambda i: (i, 0)),
            pl.BlockSpec((MBLK, 1), lambda i: (i, 0)),
            pl.BlockSpec((1, D), lambda i: (0, 0)),
        ],
        out_specs=pl.BlockSpec((MBLK, D), lambda i: (i, 0)),
        out_shape=jax.ShapeDtypeStruct((N, D), jnp.float32),
    )(agg, y1, dinv, b1)


def kernel(nodes_embeddings, edges, W0, b0, W1, b1):
    npad_e = EPAD - E
    pad_src = (jnp.arange(npad_e, dtype=jnp.int32) * 97) % N
    pad_dst = N + (jnp.arange(npad_e, dtype=jnp.int32) % (NPAD - N))
    src = jnp.concatenate(
        [edges[0].astype(jnp.int32), pad_src]).reshape(NW, CHUNKS, CK)
    dst = jnp.concatenate(
        [edges[1].astype(jnp.int32), pad_dst]).reshape(NW, CHUNKS, CK)

    zeros_deg = jnp.zeros((ZROWS, DEGW), jnp.float32)
    ones_deg = jnp.ones((CK, DEGW), jnp.float32)
    zeros_agg = jnp.zeros((ZROWS, D), jnp.float32)
    b0r = b0.reshape(1, D)
    b1r = b1.reshape(1, D)

    deg_kernel, agg_kernel = _sc_kernels()
    deg = deg_kernel(dst, zeros_deg, ones_deg)

    xw0 = _mm_call(nodes_embeddings, W0)
    dinv, y0 = _scale_call(deg, xw0)
    agg0 = agg_kernel(y0, src, dst, zeros_agg)
    y1 = _mid_call(agg0, y0, dinv, W1, b0r)
    agg1 = agg_kernel(y1, src, dst, zeros_agg)
    return _post_call(agg1, y1, dinv, b1r)
